# baseline (speedup 1.0000x reference)
import functools

import jax
import jax.numpy as jnp
from jax import lax
from jax.experimental import pallas as pl
from jax.experimental.pallas import tpu as pltpu
from jax.experimental.pallas import tpu_sc as plsc

_CH = 256


def _gather_kernel(E, V, D, num_cores, num_subcores):
    NW = num_cores * num_subcores
    b_per_w = E // NW
    n_chunks = b_per_w // _CH
    n_pairs = n_chunks // 2
    mesh = plsc.VectorSubcoreMesh(core_axis_name="c", subcore_axis_name="s")

    @functools.partial(
        pl.kernel,
        mesh=mesh,
        compiler_params=pltpu.CompilerParams(needs_layout_passes=False, disable_bounds_checks=True),
        out_type=jax.ShapeDtypeStruct((D, E), jnp.float32),
        scratch_types=[
            pltpu.VMEM((_CH,), jnp.int32),
            pltpu.VMEM((_CH,), jnp.int32),
            pltpu.VMEM((_CH,), jnp.int32),
            pltpu.VMEM((_CH,), jnp.int32),
            pltpu.VMEM((_CH, 2 * D), jnp.float32),
            pltpu.VMEM((_CH, 2 * D), jnp.float32),
            pltpu.VMEM((D, _CH), jnp.float32),
            pltpu.VMEM((D, _CH), jnp.float32),
        ]
        + [pltpu.SemaphoreType.DMA] * 4,
    )
    def k(idx_hbm, tpair_hbm, out_hbm,
          idx0, idx1, pidx0, pidx1, rows0, rows1, tbuf0, tbuf1, *sems):
        idx_v = (idx0, idx1)
        pidx_v = (pidx0, pidx1)
        rows_v = (rows0, rows1)
        tbuf_v = (tbuf0, tbuf1)
        gsems = sems[:2]
        osems = sems[2:]
        wid = lax.axis_index("s") * num_cores + lax.axis_index("c")
        base = wid * b_per_w

        def gather_start(c, b):
            pltpu.sync_copy(idx_hbm.at[pl.ds(base + c * _CH, _CH)], idx_v[b])
            for cg in range(_CH // 16):
                iv = idx_v[b][pl.ds(16 * cg, 16)]
                pidx_v[b][pl.ds(16 * cg, 16)] = iv >> 1
            pltpu.make_async_copy(
                tpair_hbm.at[pidx_v[b]], rows_v[b], gsems[b]
            ).start()

        def gather_wait(c, b):
            pltpu.make_async_copy(
                tpair_hbm.at[pidx_v[b]], rows_v[b], gsems[b]
            ).wait()

        def write_start(c, b):
            pltpu.make_async_copy(
                tbuf_v[b], out_hbm.at[:, pl.ds(base + c * _CH, _CH)], osems[b]
            ).start()

        def write_wait(c, b):
            pltpu.make_async_copy(
                tbuf_v[b], out_hbm.at[:, pl.ds(base + c * _CH, _CH)], osems[b]
            ).wait()

        def transpose(b):
            lanes = lax.iota(jnp.int32, 16)
            wraps = [(lanes + d) & 15 for d in range(16)]

            @plsc.parallel_loop(0, (_CH // 16) * (D // 16), unroll=3)
            def tbody(i):
                cg = i // (D // 16)
                fg = i % (D // 16)
                c16 = 16 * cg
                row = lanes + c16
                colbase = (idx_v[b][pl.ds(c16, 16)] & 1) << 6
                for d in range(16):
                    wf = wraps[d] + 16 * fg
                    v = plsc.load_gather(rows_v[b], [row, colbase + wf])
                    plsc.store_scatter(tbuf_v[b], [wf, row], v)

        def step(c, b, wait_prior, start_next):
            if wait_prior:
                write_wait(c - 2, b)
            if start_next:
                gather_start(c + 1, 1 - b)
            gather_wait(c, b)
            transpose(b)
            write_start(c, b)

        gather_start(0, 0)
        step(0, 0, wait_prior=False, start_next=True)
        step(1, 1, wait_prior=False, start_next=True)

        def body(g, carry):
            c0 = 2 * g
            step(c0, 0, wait_prior=True, start_next=True)
            step(c0 + 1, 1, wait_prior=True, start_next=True)
            return carry

        lax.fori_loop(1, n_pairs - 1, body, 0)

        c0 = n_chunks - 2
        step(c0, 0, wait_prior=True, start_next=True)
        step(c0 + 1, 1, wait_prior=True, start_next=False)
        write_wait(n_chunks - 2, 0)
        write_wait(n_chunks - 1, 1)

    return k


def kernel(rel_ids, emb_weight):
    E = rel_ids.shape[0]
    V, D = emb_weight.shape
    flat_ids = rel_ids.reshape(-1).astype(jnp.int32)
    tpair = emb_weight.reshape(V // 2, 2 * D)
    info = plsc.get_sparse_core_info()
    k = _gather_kernel(E, V, D, info.num_cores, info.num_subcores)
    out_t = k(flat_ids, tpair)
    return out_t.T


# speedup vs baseline: 1.0218x; 1.0218x over previous
import functools

import jax
import jax.numpy as jnp
from jax import lax
from jax.experimental import pallas as pl
from jax.experimental.pallas import tpu as pltpu
from jax.experimental.pallas import tpu_sc as plsc

_CH = 256


def _gather_kernel(E, V, D, num_cores, num_subcores):
    NW = num_cores * num_subcores
    b_per_w = E // NW
    n_chunks = b_per_w // _CH
    n_pairs = n_chunks // 2
    mesh = plsc.VectorSubcoreMesh(core_axis_name="c", subcore_axis_name="s")

    @functools.partial(
        pl.kernel,
        mesh=mesh,
        compiler_params=pltpu.CompilerParams(needs_layout_passes=False, disable_bounds_checks=True),
        out_type=jax.ShapeDtypeStruct((D, E), jnp.float32),
        scratch_types=[
            pltpu.VMEM((_CH,), jnp.int32),
            pltpu.VMEM((_CH,), jnp.int32),
            pltpu.VMEM((_CH,), jnp.int32),
            pltpu.VMEM((_CH,), jnp.int32),
            pltpu.VMEM((_CH, 2 * D), jnp.float32),
            pltpu.VMEM((_CH, 2 * D), jnp.float32),
            pltpu.VMEM((D, _CH), jnp.float32),
            pltpu.VMEM((D, _CH), jnp.float32),
        ]
        + [pltpu.SemaphoreType.DMA] * 4,
    )
    def k(idx_hbm, tpair_hbm, out_hbm,
          idx0, idx1, pidx0, pidx1, rows0, rows1, tbuf0, tbuf1, *sems):
        idx_v = (idx0, idx1)
        pidx_v = (pidx0, pidx1)
        rows_v = (rows0, rows1)
        tbuf_v = (tbuf0, tbuf1)
        gsems = sems[:2]
        osems = sems[2:]
        wid = lax.axis_index("s") * num_cores + lax.axis_index("c")
        base = wid * b_per_w

        def gather_start(c, b):
            pltpu.sync_copy(idx_hbm.at[pl.ds(base + c * _CH, _CH)], idx_v[b])
            for cg in range(_CH // 16):
                iv = idx_v[b][pl.ds(16 * cg, 16)]
                pidx_v[b][pl.ds(16 * cg, 16)] = iv >> 1
            pltpu.make_async_copy(
                tpair_hbm.at[pidx_v[b]], rows_v[b], gsems[b]
            ).start()

        def gather_wait(c, b):
            pltpu.make_async_copy(
                tpair_hbm.at[pidx_v[b]], rows_v[b], gsems[b]
            ).wait()

        def write_start(c, b):
            pltpu.make_async_copy(
                tbuf_v[b], out_hbm.at[:, pl.ds(base + c * _CH, _CH)], osems[b]
            ).start()

        def write_wait(c, b):
            pltpu.make_async_copy(
                tbuf_v[b], out_hbm.at[:, pl.ds(base + c * _CH, _CH)], osems[b]
            ).wait()

        def transpose(b):
            lanes = lax.iota(jnp.int32, 16)
            wraps = [(lanes + d) & 15 for d in range(16)]

            @plsc.parallel_loop(0, (_CH // 16) * (D // 16), unroll=2)
            def tbody(i):
                cg = i // (D // 16)
                fg = i % (D // 16)
                c16 = 16 * cg
                row = lanes + c16
                colbase = (idx_v[b][pl.ds(c16, 16)] & 1) << 6
                for d in range(16):
                    wf = wraps[d] + 16 * fg
                    v = plsc.load_gather(rows_v[b], [row, colbase + wf])
                    plsc.store_scatter(tbuf_v[b], [wf, row], v)

        def step(c, b, wait_prior, start_next):
            if wait_prior:
                write_wait(c - 2, b)
            if start_next:
                gather_start(c + 1, 1 - b)
            gather_wait(c, b)
            transpose(b)
            write_start(c, b)

        gather_start(0, 0)
        step(0, 0, wait_prior=False, start_next=True)
        step(1, 1, wait_prior=False, start_next=True)

        def body(g, carry):
            c0 = 2 * g
            step(c0, 0, wait_prior=True, start_next=True)
            step(c0 + 1, 1, wait_prior=True, start_next=True)
            return carry

        lax.fori_loop(1, n_pairs - 1, body, 0)

        c0 = n_chunks - 2
        step(c0, 0, wait_prior=True, start_next=True)
        step(c0 + 1, 1, wait_prior=True, start_next=False)
        write_wait(n_chunks - 2, 0)
        write_wait(n_chunks - 1, 1)

    return k


def kernel(rel_ids, emb_weight):
    E = rel_ids.shape[0]
    V, D = emb_weight.shape
    flat_ids = rel_ids.reshape(-1).astype(jnp.int32)
    tpair = emb_weight.reshape(V // 2, 2 * D)
    info = plsc.get_sparse_core_info()
    k = _gather_kernel(E, V, D, info.num_cores, info.num_subcores)
    out_t = k(flat_ids, tpair)
    return out_t.T
